# Initial kernel scaffold; baseline (speedup 1.0000x reference)
import jax
import jax.numpy as jnp
from jax.experimental import pallas as pl
from jax.experimental.pallas import tpu as pltpu

T = 2048
D = 1024
F = 2048
E = 8
EP = 128
BT = 256
NTB = T // BT
NEG = jnp.float32(-1e30)


def _moe_body(x_ref, Wgp_ref, bgp_ref, W1_ref, b1_ref, W2_ref, b2_ref,
              out_ref, coef_ref, w1b_ref, w2b_ref):
    e = pl.program_id(0)
    tb = pl.program_id(1)
    xb = x_ref[pl.ds(tb * BT, BT), :]

    @pl.when(e == 0)
    def _router():
        logits = jnp.dot(xb, Wgp_ref[...],
                         preferred_element_type=jnp.float32) + bgp_ref[...]
        lane = jax.lax.broadcasted_iota(jnp.int32, (BT, EP), 1)
        l = jnp.where(lane < E, logits, NEG)
        m1 = jnp.max(l, axis=1, keepdims=True)
        mask1 = l == m1
        l2 = jnp.where(mask1, NEG, l)
        m2 = jnp.max(l2, axis=1, keepdims=True)
        mask2 = l2 == m2
        g1 = 1.0 / (1.0 + jnp.exp(m2 - m1))
        g2 = 1.0 - g1
        coef_ref[pl.ds(tb * BT, BT), :] = (
            g1 * mask1.astype(jnp.float32) + g2 * mask2.astype(jnp.float32))

    @pl.when(tb == 0)
    def _cast_weights():
        w1b_ref[...] = W1_ref[0].astype(jnp.bfloat16)
        w2b_ref[...] = W2_ref[0].astype(jnp.bfloat16)

    lane = jax.lax.broadcasted_iota(jnp.int32, (BT, EP), 1)
    ce = jnp.sum(jnp.where(lane == e, coef_ref[pl.ds(tb * BT, BT), :], 0.0),
                 axis=1, keepdims=True)

    h = jnp.dot(xb.astype(jnp.bfloat16), w1b_ref[...],
                preferred_element_type=jnp.float32) + b1_ref[0]
    h = jax.nn.gelu(h)
    y = jnp.dot(h.astype(jnp.bfloat16), w2b_ref[...],
                preferred_element_type=jnp.float32) + b2_ref[0]
    contrib = ce * y

    @pl.when(e == 0)
    def _init():
        out_ref[pl.ds(tb * BT, BT), :] = contrib

    @pl.when(e > 0)
    def _acc():
        out_ref[pl.ds(tb * BT, BT), :] += contrib


def _moe_dense(xf, Wgp, bgp, W1, b1, W2, b2):
    return pl.pallas_call(
        _moe_body,
        grid=(E, NTB),
        in_specs=[
            pl.BlockSpec((T, D), lambda e, tb: (0, 0)),
            pl.BlockSpec((D, EP), lambda e, tb: (0, 0)),
            pl.BlockSpec((1, EP), lambda e, tb: (0, 0)),
            pl.BlockSpec((1, D, F), lambda e, tb: (e, 0, 0)),
            pl.BlockSpec((1, 1, F), lambda e, tb: (e, 0, 0)),
            pl.BlockSpec((1, F, D), lambda e, tb: (e, 0, 0)),
            pl.BlockSpec((1, 1, D), lambda e, tb: (e, 0, 0)),
        ],
        out_specs=pl.BlockSpec((T, D), lambda e, tb: (0, 0)),
        out_shape=jax.ShapeDtypeStruct((T, D), jnp.float32),
        scratch_shapes=[
            pltpu.VMEM((T, EP), jnp.float32),
            pltpu.VMEM((D, F), jnp.bfloat16),
            pltpu.VMEM((F, D), jnp.bfloat16),
        ],
        compiler_params=pltpu.CompilerParams(
            dimension_semantics=("arbitrary", "arbitrary")),
    )(xf, Wgp, bgp, W1, b1, W2, b2)


def kernel(x, Wg, bg, W1, b1, W2, b2):
    B, S, Dm = x.shape
    xf = x.reshape(-1, Dm)
    Wgp = jnp.zeros((D, EP), jnp.float32).at[:, :E].set(Wg)
    bgp = jnp.full((1, EP), NEG, jnp.float32).at[0, :E].set(bg)
    out = _moe_dense(xf, Wgp, bgp,
                     W1, b1.reshape(E, 1, F),
                     W2, b2.reshape(E, 1, D))
    return out.reshape(B, S, Dm)


# speedup vs baseline: 1.2013x; 1.2013x over previous
import jax
import jax.numpy as jnp
from jax.experimental import pallas as pl
from jax.experimental.pallas import tpu as pltpu

T = 2048
D = 1024
F = 2048
E = 8
EP = 128
BT = 256
NTB = T // BT
NEG = -1e30


def _moe_body(x_ref, Wgp_ref, bgp_ref, W1_ref, b1_ref, W2_ref, b2_ref,
              out_ref, coef_ref, w1b_ref, w2b_ref):
    e = pl.program_id(0)
    tb = pl.program_id(1)
    xb = x_ref[...]

    @pl.when(e == 0)
    def _router():
        logits = jnp.dot(xb, Wgp_ref[...],
                         preferred_element_type=jnp.float32) + bgp_ref[...]
        lane = jax.lax.broadcasted_iota(jnp.int32, (BT, EP), 1)
        l = jnp.where(lane < E, logits, NEG)
        m1 = jnp.max(l, axis=1, keepdims=True)
        mask1 = l == m1
        l2 = jnp.where(mask1, NEG, l)
        m2 = jnp.max(l2, axis=1, keepdims=True)
        mask2 = l2 == m2
        g1 = 1.0 / (1.0 + jnp.exp(m2 - m1))
        g2 = 1.0 - g1
        coef_ref[pl.ds(tb * BT, BT), :] = (
            g1 * mask1.astype(jnp.float32) + g2 * mask2.astype(jnp.float32))

    @pl.when(tb == 0)
    def _cast_weights():
        w1b_ref[...] = W1_ref[0].astype(jnp.bfloat16)
        w2b_ref[...] = W2_ref[0].astype(jnp.bfloat16)

    lane = jax.lax.broadcasted_iota(jnp.int32, (BT, EP), 1)
    ce = jnp.sum(jnp.where(lane == e, coef_ref[pl.ds(tb * BT, BT), :], 0.0),
                 axis=1, keepdims=True)

    h = jnp.dot(xb.astype(jnp.bfloat16), w1b_ref[...],
                preferred_element_type=jnp.float32) + b1_ref[0]
    h = jax.nn.gelu(h)
    y = jnp.dot(h.astype(jnp.bfloat16), w2b_ref[...],
                preferred_element_type=jnp.float32) + b2_ref[0]
    contrib = ce * y

    @pl.when(e == 0)
    def _init():
        out_ref[pl.ds(tb * BT, BT), :] = contrib

    @pl.when(e > 0)
    def _acc():
        out_ref[pl.ds(tb * BT, BT), :] += contrib


def _moe_dense(xf, Wgp, bgp, W1, b1, W2, b2):
    return pl.pallas_call(
        _moe_body,
        grid=(E, NTB),
        in_specs=[
            pl.BlockSpec((BT, D), lambda e, tb: (tb, 0)),
            pl.BlockSpec((D, EP), lambda e, tb: (0, 0)),
            pl.BlockSpec((1, EP), lambda e, tb: (0, 0)),
            pl.BlockSpec((1, D, F), lambda e, tb: (e, 0, 0)),
            pl.BlockSpec((1, 1, F), lambda e, tb: (e, 0, 0)),
            pl.BlockSpec((1, F, D), lambda e, tb: (e, 0, 0)),
            pl.BlockSpec((1, 1, D), lambda e, tb: (e, 0, 0)),
        ],
        out_specs=pl.BlockSpec((T, D), lambda e, tb: (0, 0)),
        out_shape=jax.ShapeDtypeStruct((T, D), jnp.float32),
        scratch_shapes=[
            pltpu.VMEM((T, EP), jnp.float32),
            pltpu.VMEM((D, F), jnp.bfloat16),
            pltpu.VMEM((F, D), jnp.bfloat16),
        ],
        compiler_params=pltpu.CompilerParams(
            dimension_semantics=("arbitrary", "arbitrary")),
    )(xf, Wgp, bgp, W1, b1, W2, b2)


def kernel(x, Wg, bg, W1, b1, W2, b2):
    B, S, Dm = x.shape
    xf = x.reshape(-1, Dm)
    Wgp = jnp.zeros((D, EP), jnp.float32).at[:, :E].set(Wg)
    bgp = jnp.full((1, EP), NEG, jnp.float32).at[0, :E].set(bg)
    out = _moe_dense(xf, Wgp, bgp,
                     W1, b1.reshape(E, 1, F),
                     W2, b2.reshape(E, 1, D))
    return out.reshape(B, S, Dm)


# speedup vs baseline: 1.7813x; 1.4829x over previous
import functools

import jax
import jax.numpy as jnp
from jax import lax
from jax.experimental import pallas as pl
from jax.experimental.pallas import tpu as pltpu
from jax.experimental.pallas import tpu_sc as plsc

T = 2048
D = 1024
F = 2048
E = 8
EP = 128
BT = 256
NSLOT = T * 2 + E * BT
NB = NSLOT // BT
NEG = -1e30

NC = 2
NS = 16
NW = NC * NS
TPW = T // NW
NV = T // 16


def _router_body(x_ref, Wgp_ref, bgp_ref, e1_ref, e2_ref, g1_ref, g2_ref):
    logitsT = lax.dot_general(
        Wgp_ref[...], x_ref[...],
        dimension_numbers=(((0,), (1,)), ((), ())),
        preferred_element_type=jnp.float32) + bgp_ref[...]
    rows = [logitsT[e:e + 1, :] for e in range(E)]
    m1 = rows[0]
    for e in range(1, E):
        m1 = jnp.maximum(m1, rows[e])
    e1 = jnp.full((1, T), E, jnp.int32)
    for e in range(E - 1, -1, -1):
        e1 = jnp.where(rows[e] == m1, e, e1)
    rows2 = [jnp.where(e1 == e, NEG, rows[e]) for e in range(E)]
    m2 = rows2[0]
    for e in range(1, E):
        m2 = jnp.maximum(m2, rows2[e])
    e2 = jnp.full((1, T), E, jnp.int32)
    for e in range(E - 1, -1, -1):
        e2 = jnp.where(rows2[e] == m2, e, e2)
    g1 = 1.0 / (1.0 + jnp.exp(m2 - m1))
    e1_ref[...] = e1
    e2_ref[...] = e2
    g1_ref[...] = g1
    g2_ref[...] = 1.0 - g1


def _router(xf, Wgp, bgp):
    return pl.pallas_call(
        _router_body,
        out_shape=(
            jax.ShapeDtypeStruct((1, T), jnp.int32),
            jax.ShapeDtypeStruct((1, T), jnp.int32),
            jax.ShapeDtypeStruct((1, T), jnp.float32),
            jax.ShapeDtypeStruct((1, T), jnp.float32),
        ),
    )(xf, Wgp, bgp)


def _iota16():
    return lax.iota(jnp.int32, 16)


def _splat(scalar):
    return jnp.full((16,), scalar, jnp.int32)


def _dispatch_body(e1_hbm, e2_hbm, x_hbm,
                   pos1_hbm, pos2_hbm, xs_hbm, bexp_hbm, meta_hbm,
                   e1_v, e2_v, pos1_v, pos2_v, rows_v, misc_v, sem):
    wid = lax.axis_index("s") * NC + lax.axis_index("c")
    w4 = wid * (TPW // 16)

    pltpu.sync_copy(e1_hbm.at[0], e1_v)
    pltpu.sync_copy(e2_hbm.at[0], e2_v)

    iota = _iota16()
    zero = jnp.zeros((16,), jnp.int32)

    def load16(ref, i):
        return ref[pl.ds(i * 16, 16)]

    def count_step(i, carry):
        accs1, accs2 = carry
        v1 = load16(e1_v, i)
        v2 = load16(e2_v, i)
        accs1 = tuple(accs1[e] + jnp.where(v1 == e, 1, 0) for e in range(E))
        accs2 = tuple(accs2[e] + jnp.where(v2 == e, 1, 0) for e in range(E))
        return accs1, accs2

    init = (tuple(zero for _ in range(E)), tuple(zero for _ in range(E)))
    pre = lax.fori_loop(0, w4, count_step, init)
    p1 = [jnp.sum(a) for a in pre[0]]
    p2 = [jnp.sum(a) for a in pre[1]]
    tot = lax.fori_loop(w4, NV, count_step, pre)
    t1 = [jnp.sum(a) for a in tot[0]]
    t2 = [jnp.sum(a) for a in tot[1]]

    totv = zero
    for e in range(E):
        totv = totv + jnp.where(iota == e, t1[e] + t2[e], 0)
    pc = ((totv + (BT - 1)) >> 8) << 8
    po_inc = plsc.cumsum(pc)
    po_exc = po_inc - pc

    po_exc_s = [jnp.sum(jnp.where(iota == e, po_exc, 0)) for e in range(E)]
    start1 = [_splat(po_exc_s[e] + p1[e]) for e in range(E)]
    start2 = [_splat(po_exc_s[e] + t1[e] + p2[e]) for e in range(E)]

    def place(src_ref, starts, pos_ref):
        for j in range(TPW // 16):
            v = load16(src_ref, w4 + j)
            pos = zero
            for e in range(E):
                m = v == e
                mi = jnp.where(m, 1, 0)
                c = plsc.cumsum(mi)
                pos = jnp.where(m, starts[e] + c - 1, pos)
                starts[e] = starts[e] + _splat(jnp.sum(mi))
            pos_ref[pl.ds(j * 16, 16)] = pos

    place(e1_v, start1, pos1_v)
    place(e2_v, start2, pos2_v)

    base = wid * TPW
    pltpu.sync_copy(pos1_v, pos1_hbm.at[pl.ds(base, TPW)])
    pltpu.sync_copy(pos2_v, pos2_hbm.at[pl.ds(base, TPW)])

    pltpu.sync_copy(x_hbm.at[pl.ds(base, TPW)], rows_v)
    cp1 = pltpu.make_async_copy(rows_v, xs_hbm.at[pos1_v], sem)
    cp1.start()
    cp2 = pltpu.make_async_copy(rows_v, xs_hbm.at[pos2_v], sem)
    cp2.start()
    cp1.wait()
    cp2.wait()

    @pl.when(wid == 0)
    def _():
        nblk = jnp.sum(jnp.where(iota == E - 1, po_inc, 0)) >> 8
        for j in range(2):
            b = iota + j * 16
            cnt = zero
            for e in range(E):
                cnt = cnt + jnp.where(b * BT >= jnp.sum(
                    jnp.where(iota == e, po_inc, 0)), 1, 0)
            misc_v[pl.ds(j * 16, 16)] = jnp.minimum(cnt, E - 1)
        misc_v[pl.ds(32, 16)] = _splat(nblk)
        pltpu.sync_copy(misc_v.at[pl.ds(0, 32)], bexp_hbm)
        pltpu.sync_copy(misc_v.at[pl.ds(32, 16)], meta_hbm)


def _dispatch(e1, e2, xf):
    mesh = plsc.VectorSubcoreMesh(core_axis_name="c", subcore_axis_name="s")
    fn = pl.kernel(
        _dispatch_body,
        out_type=(
            jax.ShapeDtypeStruct((T,), jnp.int32),
            jax.ShapeDtypeStruct((T,), jnp.int32),
            jax.ShapeDtypeStruct((NSLOT, D), jnp.float32),
            jax.ShapeDtypeStruct((32,), jnp.int32),
            jax.ShapeDtypeStruct((16,), jnp.int32),
        ),
        mesh=mesh,
        scratch_types=[
            pltpu.VMEM((T,), jnp.int32),
            pltpu.VMEM((T,), jnp.int32),
            pltpu.VMEM((TPW,), jnp.int32),
            pltpu.VMEM((TPW,), jnp.int32),
            pltpu.VMEM((TPW, D), jnp.float32),
            pltpu.VMEM((48,), jnp.int32),
            pltpu.SemaphoreType.DMA,
        ],
        compiler_params=pltpu.CompilerParams(needs_layout_passes=False),
    )
    return fn(e1, e2, xf)


def _ffn_body(bexp_ref, meta_ref, xs_ref, W1_ref, b1_ref, W2_ref, b2_ref,
              ys_ref, w1b_ref, w2b_ref):
    b = pl.program_id(0)
    nblk = meta_ref[0]

    @pl.when(b < nblk)
    def _():
        cur = bexp_ref[b]
        prev = bexp_ref[jnp.maximum(b - 1, 0)]

        @pl.when((b == 0) | (cur != prev))
        def _cast():
            w1b_ref[...] = W1_ref[0].astype(jnp.bfloat16)
            w2b_ref[...] = W2_ref[0].astype(jnp.bfloat16)

        h = jnp.dot(xs_ref[...].astype(jnp.bfloat16), w1b_ref[...],
                    preferred_element_type=jnp.float32) + b1_ref[0]
        h = jax.nn.gelu(h)
        ys_ref[...] = jnp.dot(h.astype(jnp.bfloat16), w2b_ref[...],
                              preferred_element_type=jnp.float32) + b2_ref[0]


def _ffn(bexp, meta, xs, W1, b1, W2, b2):
    def clamped(bexp_ref, meta_ref, b):
        return jnp.minimum(b, meta_ref[0] - 1)

    grid_spec = pltpu.PrefetchScalarGridSpec(
        num_scalar_prefetch=2,
        grid=(NB,),
        in_specs=[
            pl.BlockSpec((BT, D),
                         lambda b, be, me: (clamped(be, me, b), 0)),
            pl.BlockSpec((1, D, F),
                         lambda b, be, me: (be[clamped(be, me, b)], 0, 0)),
            pl.BlockSpec((1, 1, F),
                         lambda b, be, me: (be[clamped(be, me, b)], 0, 0)),
            pl.BlockSpec((1, F, D),
                         lambda b, be, me: (be[clamped(be, me, b)], 0, 0)),
            pl.BlockSpec((1, 1, D),
                         lambda b, be, me: (be[clamped(be, me, b)], 0, 0)),
        ],
        out_specs=pl.BlockSpec((BT, D),
                               lambda b, be, me: (clamped(be, me, b), 0)),
        scratch_shapes=[
            pltpu.VMEM((D, F), jnp.bfloat16),
            pltpu.VMEM((F, D), jnp.bfloat16),
        ],
    )
    return pl.pallas_call(
        _ffn_body,
        grid_spec=grid_spec,
        out_shape=jax.ShapeDtypeStruct((NSLOT, D), jnp.float32),
        compiler_params=pltpu.CompilerParams(
            dimension_semantics=("arbitrary",)),
    )(bexp, meta, xs, W1, b1, W2, b2)


CHUNK = 32


def _combine_body(ys_hbm, pos1_hbm, pos2_hbm, g1_hbm, g2_hbm, out_hbm,
                  idx1_v, idx2_v, g1_v, g2_v, rows1_v, rows2_v,
                  sem1, sem2):
    wid = lax.axis_index("s") * NC + lax.axis_index("c")
    iota = _iota16()

    for half in range(TPW // CHUNK):
        base = wid * TPW + half * CHUNK
        pltpu.sync_copy(pos1_hbm.at[pl.ds(base, CHUNK)], idx1_v)
        pltpu.sync_copy(pos2_hbm.at[pl.ds(base, CHUNK)], idx2_v)
        pltpu.sync_copy(g1_hbm.at[0, pl.ds(base, CHUNK)],
                        g1_v.at[pl.ds(0, CHUNK)])
        pltpu.sync_copy(g2_hbm.at[0, pl.ds(base, CHUNK)],
                        g2_v.at[pl.ds(0, CHUNK)])
        cp1 = pltpu.make_async_copy(ys_hbm.at[idx1_v], rows1_v, sem1)
        cp1.start()
        cp2 = pltpu.make_async_copy(ys_hbm.at[idx2_v], rows2_v, sem2)
        cp2.start()
        cp1.wait()
        cp2.wait()

        def row(j, _):
            w1 = g1_v[pl.ds(j, 16)]
            w2 = g2_v[pl.ds(j, 16)]
            a1 = plsc.cumsum(jnp.where(iota == 0, w1, 0.0))
            a2 = plsc.cumsum(jnp.where(iota == 0, w2, 0.0))
            for q in range(D // 16):
                r1 = rows1_v[j, pl.ds(q * 16, 16)]
                r2 = rows2_v[j, pl.ds(q * 16, 16)]
                rows1_v[j, pl.ds(q * 16, 16)] = a1 * r1 + a2 * r2
            return 0

        lax.fori_loop(0, CHUNK, row, 0)
        pltpu.sync_copy(rows1_v, out_hbm.at[pl.ds(base, CHUNK)])


def _combine(ys, pos1, pos2, g1, g2):
    mesh = plsc.VectorSubcoreMesh(core_axis_name="c", subcore_axis_name="s")
    fn = pl.kernel(
        _combine_body,
        out_type=jax.ShapeDtypeStruct((T, D), jnp.float32),
        mesh=mesh,
        scratch_types=[
            pltpu.VMEM((CHUNK,), jnp.int32),
            pltpu.VMEM((CHUNK,), jnp.int32),
            pltpu.VMEM((CHUNK + 16,), jnp.float32),
            pltpu.VMEM((CHUNK + 16,), jnp.float32),
            pltpu.VMEM((CHUNK, D), jnp.float32),
            pltpu.VMEM((CHUNK, D), jnp.float32),
            pltpu.SemaphoreType.DMA,
            pltpu.SemaphoreType.DMA,
        ],
        compiler_params=pltpu.CompilerParams(needs_layout_passes=False),
    )
    return fn(ys, pos1, pos2, g1, g2)


def kernel(x, Wg, bg, W1, b1, W2, b2):
    B, S, Dm = x.shape
    xf = x.reshape(-1, Dm)
    Wgp = jnp.zeros((D, EP), jnp.float32).at[:, :E].set(Wg)
    bgp = jnp.full((EP, 1), NEG, jnp.float32).at[:E, 0].set(bg)
    e1, e2, g1, g2 = _router(xf, Wgp, bgp)
    pos1, pos2, xs, bexp, meta = _dispatch(e1, e2, xf)
    ys = _ffn(bexp, meta, xs, W1, b1.reshape(E, 1, F), W2,
              b2.reshape(E, 1, D))
    out = _combine(ys, pos1, pos2, g1, g2)
    return out.reshape(B, S, Dm)
